# baseline (speedup 1.0000x reference)
import functools

import jax
import jax.numpy as jnp
from jax import lax
from jax.experimental import pallas as pl
from jax.experimental.pallas import tpu as pltpu
from jax.experimental.pallas import tpu_sc as plsc

B = 16384
V = 100000
D = 64
V0 = 0.5

_info = plsc.get_sparse_core_info()
_NC, _NS, _L = _info.num_cores, _info.num_subcores, _info.num_lanes
_NW = _NC * _NS
_BPW = B // _NW


def _gather_body(t1_hbm, i1_hbm, t2_hbm, i2_hbm, e1_hbm, e2_hbm,
                 idx1_v, idx2_v, rows1_v, rows2_v, sem1, sem2):
    wid = lax.axis_index("s") * _NC + lax.axis_index("c")
    base = wid * _BPW
    pltpu.sync_copy(i1_hbm.at[pl.ds(base, _BPW)], idx1_v)
    pltpu.sync_copy(i2_hbm.at[pl.ds(base, _BPW)], idx2_v)
    cp1 = pltpu.async_copy(t1_hbm.at[idx1_v], rows1_v, sem1)
    cp2 = pltpu.async_copy(t2_hbm.at[idx2_v], rows2_v, sem2)
    cp1.wait()
    pltpu.sync_copy(rows1_v, e1_hbm.at[pl.ds(base, _BPW)])
    cp2.wait()
    pltpu.sync_copy(rows2_v, e2_hbm.at[pl.ds(base, _BPW)])


@jax.jit
def _sc_gather(table1, item1, table2, item2):
    mesh = plsc.VectorSubcoreMesh(core_axis_name="c", subcore_axis_name="s")
    f = functools.partial(
        pl.kernel,
        mesh=mesh,
        compiler_params=pltpu.CompilerParams(use_tc_tiling_on_sc=False),
        out_type=[
            jax.ShapeDtypeStruct((B, D), jnp.float32),
            jax.ShapeDtypeStruct((B, D), jnp.float32),
        ],
        scratch_types=[
            pltpu.VMEM((_BPW,), jnp.int32),
            pltpu.VMEM((_BPW,), jnp.int32),
            pltpu.VMEM((_BPW, D), jnp.float32),
            pltpu.VMEM((_BPW, D), jnp.float32),
            pltpu.SemaphoreType.DMA,
            pltpu.SemaphoreType.DMA,
        ],
    )(_gather_body)
    return f(table1, item1, table2, item2)


_BLK = 2048


def _dense_body(e1_ref, e2_ref, v1_ref, v2_ref, w_ref, o_ref):
    w1 = w_ref[0:D, :]
    w2 = w_ref[D:2 * D, :]
    e1 = e1_ref[...]
    e2 = e2_ref[...]
    a = e1 - e2
    b = e1 * (v1_ref[...] - V0) - e2 * (v2_ref[...] - V0)
    o_ref[...] = (jnp.dot(a, w1, preferred_element_type=jnp.float32)
                  + jnp.dot(b, w2, preferred_element_type=jnp.float32))


@jax.jit
def _dense(e1, e2, value1, value2, W):
    grid = (B // _BLK,)
    return pl.pallas_call(
        _dense_body,
        grid=grid,
        in_specs=[
            pl.BlockSpec((_BLK, D), lambda i: (i, 0)),
            pl.BlockSpec((_BLK, D), lambda i: (i, 0)),
            pl.BlockSpec((_BLK, 1), lambda i: (i, 0)),
            pl.BlockSpec((_BLK, 1), lambda i: (i, 0)),
            pl.BlockSpec((2 * D, 1), lambda i: (0, 0)),
        ],
        out_specs=pl.BlockSpec((_BLK, 1), lambda i: (i, 0)),
        out_shape=jax.ShapeDtypeStruct((B, 1), jnp.float32),
    )(e1, e2, value1, value2, W)


def kernel(item1, value1, item2, value2, table1, table2, W):
    i1 = item1.astype(jnp.int32)
    i2 = item2.astype(jnp.int32)
    e1, e2 = _sc_gather(table1, i1, table2, i2)
    return _dense(e1, e2, value1, value2, W)
